# baseline (speedup 1.0000x reference)
import functools

import jax
import jax.numpy as jnp
from jax import lax
from jax.experimental import pallas as pl
from jax.experimental.pallas import tpu as pltpu
from jax.experimental.pallas import tpu_sc as plsc

_B, _C, _N, _K = 1, 128, 10000, 32
_OUT = 128
_NTILES = 32
_CPT = _C // _NTILES
_CHUNK = 400
_NGROUPS = _CHUNK // 16
_NCHUNKS = _N // _CHUNK
_NEG = -1e30


_POS_PAIR = 0x7F7F7F7F
_NEG_PAIR = 0xFF7FFF7F - 0x100000000
_CPP = _CPT // 2


_SEG = 2000


def _sc_max_relative(xt, et):
    mesh = plsc.VectorSubcoreMesh(core_axis_name="c", subcore_axis_name="s")

    @functools.partial(
        pl.kernel,
        out_type=jax.ShapeDtypeStruct((_C // 2, _N), jnp.int32),
        mesh=mesh,
        scratch_types=[
            pltpu.VMEM((_CPP, _N), jnp.int32),
            pltpu.VMEM((2, 2, _K, _CHUNK), jnp.int32),
            pltpu.VMEM((2, _CPP, _CHUNK), jnp.int32),
            pltpu.VMEM((2 * _CPP, _SEG), jnp.float32),
            pltpu.SemaphoreType.DMA((2,)),
            pltpu.SemaphoreType.DMA((2,)),
        ],
        compiler_params=pltpu.CompilerParams(
            use_tc_tiling_on_sc=False, needs_layout_passes=False),
    )
    def sc_kernel(xt_hbm, et_hbm, out_hbm, xt_v, idx_v, out_v, row_v, sems,
                  osems):
        wid = lax.axis_index("s") * 2 + lax.axis_index("c")
        r0 = wid * _CPP

        def start_fetch(ci, buf):
            col = ci * _CHUNK
            pltpu.async_copy(
                et_hbm.at[0, :, pl.ds(col, _CHUNK)], idx_v.at[buf, 0],
                sems.at[buf])
            pltpu.async_copy(
                et_hbm.at[1, :, pl.ds(col, _CHUNK)], idx_v.at[buf, 1],
                sems.at[buf])

        def wait_fetch(buf):
            for h in range(2):
                pltpu.make_async_copy(
                    et_hbm.at[h, :, pl.ds(0, _CHUNK)], idx_v.at[buf, h],
                    sems.at[buf],
                ).wait()

        start_fetch(0, 0)

        for seg in range(_N // _SEG):
            s0 = seg * _SEG
            pltpu.sync_copy(
                xt_hbm.at[pl.ds(r0, _CPP), pl.ds(s0, _SEG)],
                row_v.at[pl.ds(0, _CPP)])
            pltpu.sync_copy(
                xt_hbm.at[pl.ds(_C // 2 + r0, _CPP), pl.ds(s0, _SEG)],
                row_v.at[pl.ds(_CPP, _CPP)])

            @plsc.parallel_loop(0, _SEG // 16)
            def pack_body(v):
                off = v * 16
                for cp in range(_CPP):
                    packed = plsc.pack(
                        row_v[cp, pl.ds(off, 16)],
                        row_v[_CPP + cp, pl.ds(off, 16)],
                        format=plsc.PackFormat.INTERLEAVED)
                    xt_v[cp, pl.ds(s0 + off, 16)] = plsc.bitcast(
                        packed, jnp.int32)

        def wait_out(buf):
            pltpu.make_async_copy(
                out_v.at[buf],
                out_hbm.at[pl.ds(0, _CPP), pl.ds(0, _CHUNK)],
                osems.at[buf],
            ).wait()

        def chunk_compute(ci, buf):
            col = ci * _CHUNK

            @pl.when(ci >= 2)
            def _():
                wait_out(buf)

            @plsc.parallel_loop(0, _NGROUPS)
            def group_body(g):
                base = g * 16
                neg = plsc.bitcast(
                    jnp.full((16,), _NEG_PAIR, jnp.int32), jnp.bfloat16)

                def k_body(k8, accs):
                    acc0, acc1 = accs
                    for kk8 in range(8):
                        kk = k8 * 8 + kk8
                        i0 = idx_v[buf, 0, kk, pl.ds(base, 16)]
                        i1 = idx_v[buf, 1, kk, pl.ds(base, 16)]
                        valid = i0 != i1
                        cap = plsc.bitcast(
                            jnp.where(valid, jnp.int32(_POS_PAIR),
                                      jnp.int32(_NEG_PAIR)),
                            jnp.bfloat16)
                        for cp in range(_CPP):
                            xj = plsc.bitcast(
                                plsc.load_gather(xt_v.at[cp], [i0]),
                                jnp.bfloat16)
                            xi = plsc.bitcast(
                                plsc.load_gather(xt_v.at[cp], [i1]),
                                jnp.bfloat16)
                            d = jnp.minimum(xj - xi, cap)
                            if cp == 0:
                                acc0 = jnp.maximum(acc0, d)
                            else:
                                acc1 = jnp.maximum(acc1, d)
                    return acc0, acc1

                accs = lax.fori_loop(0, _K // 8, k_body, (neg, neg))
                for cp in range(_CPP):
                    out_v[buf, cp, pl.ds(base, 16)] = plsc.bitcast(
                        accs[cp], jnp.int32)

            pltpu.async_copy(
                out_v.at[buf],
                out_hbm.at[pl.ds(r0, _CPP), pl.ds(col, _CHUNK)],
                osems.at[buf])

        def pair_body(p, carry):
            for b in range(2):
                ci = 2 * p + b

                @pl.when(ci < _NCHUNKS)
                def _():
                    @pl.when(ci + 1 < _NCHUNKS)
                    def _():
                        start_fetch(ci + 1, 1 - b)

                    wait_fetch(b)
                    chunk_compute(ci, b)

            return carry

        lax.fori_loop(0, (_NCHUNKS + 1) // 2, pair_body, 0)
        wait_out(0)
        wait_out(1)

    return sc_kernel(xt, et)


_BN = 2048


def _tc_xconv(xt, W1, b2):
    def body(w_ref, b_ref, x_ref, o_ref):
        o_ref[...] = jnp.dot(w_ref[...], x_ref[...],
                             preferred_element_type=jnp.float32) + b_ref[...]

    return pl.pallas_call(
        body,
        grid=(pl.cdiv(_N, _BN),),
        in_specs=[
            pl.BlockSpec((_OUT, _C), lambda i: (0, 0)),
            pl.BlockSpec((_OUT, 1), lambda i: (0, 0)),
            pl.BlockSpec((_C, _BN), lambda i: (0, i)),
        ],
        out_specs=pl.BlockSpec((_OUT, _BN), lambda i: (0, i)),
        out_shape=jax.ShapeDtypeStruct((_OUT, _N), jnp.float32),
    )(W1, b2, xt)


def _tc_mconv(p1, mp, W2):
    def body(w_ref, p_ref, m_ref, o_ref):
        words = m_ref[...]
        lo = lax.bitcast_convert_type(words << 16, jnp.float32)
        hi = lax.bitcast_convert_type(
            words & jnp.int32(-65536), jnp.float32)
        m_full = jnp.maximum(jnp.concatenate([lo, hi], axis=0), _NEG)
        acc = jnp.dot(w_ref[...], m_full,
                      preferred_element_type=jnp.float32)
        o_ref[...] = jnp.maximum(acc + p_ref[...], 0.0)

    return pl.pallas_call(
        body,
        grid=(pl.cdiv(_N, _BN),),
        in_specs=[
            pl.BlockSpec((_OUT, _C), lambda i: (0, 0)),
            pl.BlockSpec((_OUT, _BN), lambda i: (0, i)),
            pl.BlockSpec((_C // 2, _BN), lambda i: (0, i)),
        ],
        out_specs=pl.BlockSpec((_OUT, _BN), lambda i: (0, i)),
        out_shape=jax.ShapeDtypeStruct((_OUT, _N), jnp.float32),
    )(W2, p1, mp)


def kernel(x, x_0, W, b, edge_index):
    xt = x[0, :, :, 0]
    e2 = edge_index.astype(jnp.int32).reshape(2, _N, _K)
    et = jnp.transpose(e2, (0, 2, 1))
    m = _sc_max_relative(xt, et)
    p1 = _tc_xconv(xt, W[:, :_C], b.reshape(_OUT, 1))
    out = _tc_mconv(p1, m, W[:, _C:])
    return out[None, :, :, None]
